# baseline (speedup 1.0000x reference)
import functools

import jax
import jax.numpy as jnp
from jax import lax
from jax.experimental import pallas as pl
from jax.experimental.pallas import tpu as pltpu
from jax.experimental.pallas import tpu_sc as plsc

N = 10000
E = 320000
D_IN = 128
D_HID = 64
D_OUT = 10

NC = 2
NS = 16
NW = NC * NS

K = 128
NBLK = 80
GB = 4
NG = NBLK // GB
EPT = NBLK * K
E_PAD = EPT * NW
NP = 10240
RPT = NP // NS

B = 1000
NB = N // B

_MESH = plsc.VectorSubcoreMesh(core_axis_name="c", subcore_axis_name="s")
_SC_PARAMS = pltpu.CompilerParams(use_tc_tiling_on_sc=False)


def _sc_degree(edg4, zrow):
    @functools.partial(
        pl.kernel,
        out_type=jax.ShapeDtypeStruct((NC, NP), jnp.float32),
        mesh=_MESH,
        compiler_params=_SC_PARAMS,
        scratch_types=[
            pltpu.VMEM((NBLK, K), jnp.int32),
            pltpu.VMEM((K,), jnp.float32),
            pltpu.VMEM_SHARED((NP,), jnp.float32),
        ],
    )
    def k(edg_hbm, z_hbm, out_hbm, didx, ones_v, deg_sh):
        c = lax.axis_index("c")
        s = lax.axis_index("s")
        w = c * NS + s
        for i in range(K // 16):
            ones_v[pl.ds(i * 16, 16)] = jnp.ones((16,), jnp.float32)
        pltpu.sync_copy(z_hbm, deg_sh.at[pl.ds(s * RPT, RPT)])
        pltpu.sync_copy(edg_hbm.at[1, w], didx)
        plsc.subcore_barrier()

        def body(b, carry):
            pltpu.sync_copy(ones_v, deg_sh.at[didx.at[b]], add=True)
            return carry

        lax.fori_loop(0, NBLK, body, 0)
        plsc.subcore_barrier()
        pltpu.sync_copy(deg_sh.at[pl.ds(s * RPT, RPT)],
                        out_hbm.at[c, pl.ds(s * RPT, RPT)])

    return k(edg4, zrow)


def _sc_gather_scatter(y, edg4, zrows, d):
    @functools.partial(
        pl.kernel,
        out_type=jax.ShapeDtypeStruct((NC, NP, d), jnp.float32),
        mesh=_MESH,
        compiler_params=_SC_PARAMS,
        scratch_types=[
            pltpu.VMEM((NBLK, K), jnp.int32),
            pltpu.VMEM((NBLK, K), jnp.int32),
            pltpu.VMEM((2 * GB, K, d), jnp.float32),
            pltpu.VMEM_SHARED((NP, d), jnp.float32),
            pltpu.SemaphoreType.DMA,
            pltpu.SemaphoreType.DMA,
        ],
    )
    def k(y_hbm, edg_hbm, z_hbm, out_hbm, sidx, didx, rows, agg, gsem, ssem):
        c = lax.axis_index("c")
        s = lax.axis_index("s")
        w = c * NS + s
        pltpu.sync_copy(z_hbm, agg.at[pl.ds(s * RPT, RPT)])
        pltpu.sync_copy(edg_hbm.at[0, w], sidx)
        pltpu.sync_copy(edg_hbm.at[1, w], didx)
        plsc.subcore_barrier()

        def body(g, carry):
            @pl.when(g < NG)
            def _fire():
                h = (g % 2) * GB
                for j in range(GB):
                    pltpu.async_copy(
                        y_hbm.at[sidx.at[g * GB + j]], rows.at[h + j], gsem)

            @pl.when(g > 0)
            def _drain():
                p = g - 1
                h = (p % 2) * GB
                for j in range(GB):
                    pltpu.make_async_copy(
                        y_hbm.at[sidx.at[p * GB + j]], rows.at[h + j],
                        gsem).wait()
                for j in range(GB):
                    pltpu.async_copy(
                        rows.at[h + j], agg.at[didx.at[p * GB + j]], ssem,
                        add=True)
                for j in range(GB):
                    pltpu.make_async_copy(
                        rows.at[h + j], agg.at[didx.at[p * GB + j]],
                        ssem).wait()

            return carry

        lax.fori_loop(0, NG + 1, body, 0)
        plsc.subcore_barrier()
        pltpu.sync_copy(agg.at[pl.ds(s * RPT, RPT)],
                        out_hbm.at[c, pl.ds(s * RPT, RPT)])

    return k(y, edg4, zrows)


def _deg_specs():
    return [
        pl.BlockSpec((1, B, 1), lambda r: (0, r, 0)),
        pl.BlockSpec((1, B, 1), lambda r: (1, r, 0)),
    ]


def _tc1(features, W1, deg):
    def body(f_ref, w_ref, d0_ref, d1_ref, y_ref):
        dinv = lax.rsqrt(d0_ref[0] + d1_ref[0] + 1.0)
        xw = jnp.dot(f_ref[...], w_ref[...], preferred_element_type=jnp.float32)
        y_ref[...] = xw * dinv

    d0s, d1s = _deg_specs()
    return pl.pallas_call(
        body,
        grid=(NB,),
        in_specs=[
            pl.BlockSpec((B, D_IN), lambda r: (r, 0)),
            pl.BlockSpec((D_IN, D_HID), lambda r: (0, 0)),
            d0s, d1s,
        ],
        out_specs=pl.BlockSpec((B, D_HID), lambda r: (r, 0)),
        out_shape=jax.ShapeDtypeStruct((N, D_HID), jnp.float32),
    )(features, W1, deg, deg)


def _tc2(agg1, y1, deg, b1, W2p):
    def body(a0_ref, a1_ref, y1_ref, d0_ref, d1_ref, b1_ref, w2_ref, y2_ref):
        dinv = lax.rsqrt(d0_ref[0] + d1_ref[0] + 1.0)
        h1 = jnp.maximum(
            dinv * (a0_ref[0] + a1_ref[0] + y1_ref[...]) + b1_ref[...], 0.0)
        w2p = jnp.concatenate(
            [w2_ref[...], jnp.zeros((D_HID, 16 - D_OUT), jnp.float32)], axis=1)
        xw2 = jnp.dot(h1, w2p, preferred_element_type=jnp.float32)
        y2_ref[...] = xw2 * dinv

    d0s, d1s = _deg_specs()
    return pl.pallas_call(
        body,
        grid=(NB,),
        in_specs=[
            pl.BlockSpec((1, B, D_HID), lambda r: (0, r, 0)),
            pl.BlockSpec((1, B, D_HID), lambda r: (1, r, 0)),
            pl.BlockSpec((B, D_HID), lambda r: (r, 0)),
            d0s, d1s,
            pl.BlockSpec((1, D_HID), lambda r: (0, 0)),
            pl.BlockSpec((D_HID, D_OUT), lambda r: (0, 0)),
        ],
        out_specs=pl.BlockSpec((B, 16), lambda r: (r, 0)),
        out_shape=jax.ShapeDtypeStruct((N, 16), jnp.float32),
    )(agg1, agg1, y1, deg, deg, b1, W2p)


def _tc3(agg2, y2, deg, b2p):
    def body(a0_ref, a1_ref, y2_ref, d0_ref, d1_ref, b2_ref, o_ref):
        dinv = lax.rsqrt(d0_ref[0] + d1_ref[0] + 1.0)
        b2p = jnp.concatenate(
            [b2_ref[...], jnp.zeros((1, 16 - D_OUT), jnp.float32)], axis=1)
        z = dinv * (a0_ref[0] + a1_ref[0] + y2_ref[...]) + b2p
        col = lax.broadcasted_iota(jnp.int32, (B, 16), 1)
        zm = jnp.where(col < D_OUT, z, -1e30)
        m = jnp.max(zm, axis=1, keepdims=True)
        lse = jnp.log(jnp.sum(jnp.exp(zm - m), axis=1, keepdims=True)) + m
        o_ref[...] = (z - lse)[:, :D_OUT]

    d0s, d1s = _deg_specs()
    return pl.pallas_call(
        body,
        grid=(NB,),
        in_specs=[
            pl.BlockSpec((1, B, 16), lambda r: (0, r, 0)),
            pl.BlockSpec((1, B, 16), lambda r: (1, r, 0)),
            pl.BlockSpec((B, 16), lambda r: (r, 0)),
            d0s, d1s,
            pl.BlockSpec((1, D_OUT), lambda r: (0, 0)),
        ],
        out_specs=pl.BlockSpec((B, D_OUT), lambda r: (r, 0)),
        out_shape=jax.ShapeDtypeStruct((N, D_OUT), jnp.float32),
    )(agg2, agg2, y2, deg, deg, b2p)


def kernel(features, edges, W1, b1, W2, b2):
    pad = E_PAD - E
    pad_idx = lax.iota(jnp.int32, pad)
    pad2 = jnp.stack([pad_idx % 4096, N + pad_idx % (NP - N)])
    edg4 = jnp.concatenate([edges, pad2], axis=1).reshape(2, NW, NBLK, K)

    z1 = jnp.zeros((RPT,), jnp.float32)
    z64 = jnp.zeros((RPT, D_HID), jnp.float32)
    z16 = jnp.zeros((RPT, 16), jnp.float32)
    b1r = b1.reshape(1, D_HID)
    b2r = b2.reshape(1, D_OUT)

    deg = _sc_degree(edg4, z1)[:, :, None]
    y1 = _tc1(features, W1, deg)
    agg1 = _sc_gather_scatter(y1, edg4, z64, D_HID)
    y2 = _tc2(agg1, y1, deg, b1r, W2)
    agg2 = _sc_gather_scatter(y2, edg4, z16, 16)
    return _tc3(agg2, y2, deg, b2r)


# speedup vs baseline: 1.0177x; 1.0177x over previous
import functools

import jax
import jax.numpy as jnp
from jax import lax
from jax.experimental import pallas as pl
from jax.experimental.pallas import tpu as pltpu
from jax.experimental.pallas import tpu_sc as plsc

N = 10000
E = 320000
D_IN = 128
D_HID = 64
D_OUT = 10

NC = 2
NS = 16
NW = NC * NS

K = 128
NBLK = 80
GB = 5
NG = NBLK // GB
EPT = NBLK * K
E_PAD = EPT * NW
NP = 10240
RPT = NP // NS

B = 1000
NB = N // B

_MESH = plsc.VectorSubcoreMesh(core_axis_name="c", subcore_axis_name="s")
_SC_PARAMS = pltpu.CompilerParams(use_tc_tiling_on_sc=False)


def _sc_degree(edg4, zrow):
    @functools.partial(
        pl.kernel,
        out_type=jax.ShapeDtypeStruct((NC, NP), jnp.float32),
        mesh=_MESH,
        compiler_params=_SC_PARAMS,
        scratch_types=[
            pltpu.VMEM((NBLK, K), jnp.int32),
            pltpu.VMEM((K,), jnp.float32),
            pltpu.VMEM_SHARED((NP,), jnp.float32),
        ],
    )
    def k(edg_hbm, z_hbm, out_hbm, didx, ones_v, deg_sh):
        c = lax.axis_index("c")
        s = lax.axis_index("s")
        w = c * NS + s
        for i in range(K // 16):
            ones_v[pl.ds(i * 16, 16)] = jnp.ones((16,), jnp.float32)
        pltpu.sync_copy(z_hbm, deg_sh.at[pl.ds(s * RPT, RPT)])
        pltpu.sync_copy(edg_hbm.at[1, w], didx)
        plsc.subcore_barrier()

        def body(b, carry):
            pltpu.sync_copy(ones_v, deg_sh.at[didx.at[b]], add=True)
            return carry

        lax.fori_loop(0, NBLK, body, 0)
        plsc.subcore_barrier()
        pltpu.sync_copy(deg_sh.at[pl.ds(s * RPT, RPT)],
                        out_hbm.at[c, pl.ds(s * RPT, RPT)])

    return k(edg4, zrow)


def _sc_gather_scatter(y, edg4, zrows, d):
    @functools.partial(
        pl.kernel,
        out_type=jax.ShapeDtypeStruct((NC, NP, d), jnp.float32),
        mesh=_MESH,
        compiler_params=_SC_PARAMS,
        scratch_types=[
            pltpu.VMEM((4 * GB, K), jnp.int32),
            pltpu.VMEM((4 * GB, K), jnp.int32),
            pltpu.VMEM((2 * GB, K, d), jnp.float32),
            pltpu.VMEM_SHARED((NP, d), jnp.float32),
            pltpu.SemaphoreType.DMA,
            pltpu.SemaphoreType.DMA,
            pltpu.SemaphoreType.DMA,
        ],
    )
    def k(y_hbm, edg_hbm, z_hbm, out_hbm, sidx, didx, rows, agg,
          gsem, ssem, isem):
        c = lax.axis_index("c")
        s = lax.axis_index("s")
        w = c * NS + s
        pltpu.sync_copy(z_hbm, agg.at[pl.ds(s * RPT, RPT)])
        pltpu.sync_copy(edg_hbm.at[0, w, pl.ds(0, 2 * GB)],
                        sidx.at[pl.ds(0, 2 * GB)])
        pltpu.sync_copy(edg_hbm.at[1, w, pl.ds(0, 2 * GB)],
                        didx.at[pl.ds(0, 2 * GB)])
        plsc.subcore_barrier()

        def _idx_copies(g):
            slot = (g % 4) * GB
            return (
                pltpu.make_async_copy(
                    edg_hbm.at[0, w, pl.ds(g * GB, GB)],
                    sidx.at[pl.ds(slot, GB)], isem),
                pltpu.make_async_copy(
                    edg_hbm.at[1, w, pl.ds(g * GB, GB)],
                    didx.at[pl.ds(slot, GB)], isem),
            )

        def body(g, carry):
            @pl.when(jnp.logical_and(g >= 2, g - 2 < NG))
            def _drain_scatters():
                p = g - 2
                h = (p % 2) * GB
                for j in range(GB):
                    pltpu.make_async_copy(
                        rows.at[h + j], agg.at[didx.at[j]], ssem).wait()

            @pl.when(g + 2 < NG)
            def _fire_idx():
                for cp in _idx_copies(g + 2):
                    cp.start()

            @pl.when(jnp.logical_and(g >= 2, g < NG))
            def _wait_idx():
                for cp in _idx_copies(g):
                    cp.wait()

            @pl.when(g < NG)
            def _fire_gathers():
                slot = (g % 4) * GB
                h = (g % 2) * GB
                for j in range(GB):
                    pltpu.async_copy(
                        y_hbm.at[sidx.at[slot + j]], rows.at[h + j], gsem)

            @pl.when(jnp.logical_and(g >= 1, g - 1 < NG))
            def _scatter_prev():
                p = g - 1
                slot = (p % 4) * GB
                h = (p % 2) * GB
                for j in range(GB):
                    pltpu.make_async_copy(
                        y_hbm.at[sidx.at[slot + j]], rows.at[h + j],
                        gsem).wait()
                for j in range(GB):
                    pltpu.async_copy(
                        rows.at[h + j], agg.at[didx.at[slot + j]], ssem,
                        add=True)

            return carry

        lax.fori_loop(0, NG + 2, body, 0)
        plsc.subcore_barrier()
        pltpu.sync_copy(agg.at[pl.ds(s * RPT, RPT)],
                        out_hbm.at[c, pl.ds(s * RPT, RPT)])

    return k(y, edg4, zrows)


def _deg_specs():
    return [
        pl.BlockSpec((1, B, 1), lambda r: (0, r, 0)),
        pl.BlockSpec((1, B, 1), lambda r: (1, r, 0)),
    ]


def _tc1(features, W1, deg):
    def body(f_ref, w_ref, d0_ref, d1_ref, y_ref):
        dinv = lax.rsqrt(d0_ref[0] + d1_ref[0] + 1.0)
        xw = jnp.dot(f_ref[...], w_ref[...], preferred_element_type=jnp.float32)
        y_ref[...] = xw * dinv

    d0s, d1s = _deg_specs()
    return pl.pallas_call(
        body,
        grid=(NB,),
        in_specs=[
            pl.BlockSpec((B, D_IN), lambda r: (r, 0)),
            pl.BlockSpec((D_IN, D_HID), lambda r: (0, 0)),
            d0s, d1s,
        ],
        out_specs=pl.BlockSpec((B, D_HID), lambda r: (r, 0)),
        out_shape=jax.ShapeDtypeStruct((N, D_HID), jnp.float32),
    )(features, W1, deg, deg)


def _tc2(agg1, y1, deg, b1, W2p):
    def body(a0_ref, a1_ref, y1_ref, d0_ref, d1_ref, b1_ref, w2_ref, y2_ref):
        dinv = lax.rsqrt(d0_ref[0] + d1_ref[0] + 1.0)
        h1 = jnp.maximum(
            dinv * (a0_ref[0] + a1_ref[0] + y1_ref[...]) + b1_ref[...], 0.0)
        w2p = jnp.concatenate(
            [w2_ref[...], jnp.zeros((D_HID, 16 - D_OUT), jnp.float32)], axis=1)
        xw2 = jnp.dot(h1, w2p, preferred_element_type=jnp.float32)
        y2_ref[...] = xw2 * dinv

    d0s, d1s = _deg_specs()
    return pl.pallas_call(
        body,
        grid=(NB,),
        in_specs=[
            pl.BlockSpec((1, B, D_HID), lambda r: (0, r, 0)),
            pl.BlockSpec((1, B, D_HID), lambda r: (1, r, 0)),
            pl.BlockSpec((B, D_HID), lambda r: (r, 0)),
            d0s, d1s,
            pl.BlockSpec((1, D_HID), lambda r: (0, 0)),
            pl.BlockSpec((D_HID, D_OUT), lambda r: (0, 0)),
        ],
        out_specs=pl.BlockSpec((B, 16), lambda r: (r, 0)),
        out_shape=jax.ShapeDtypeStruct((N, 16), jnp.float32),
    )(agg1, agg1, y1, deg, deg, b1, W2p)


def _tc3(agg2, y2, deg, b2p):
    def body(a0_ref, a1_ref, y2_ref, d0_ref, d1_ref, b2_ref, o_ref):
        dinv = lax.rsqrt(d0_ref[0] + d1_ref[0] + 1.0)
        b2p = jnp.concatenate(
            [b2_ref[...], jnp.zeros((1, 16 - D_OUT), jnp.float32)], axis=1)
        z = dinv * (a0_ref[0] + a1_ref[0] + y2_ref[...]) + b2p
        col = lax.broadcasted_iota(jnp.int32, (B, 16), 1)
        zm = jnp.where(col < D_OUT, z, -1e30)
        m = jnp.max(zm, axis=1, keepdims=True)
        lse = jnp.log(jnp.sum(jnp.exp(zm - m), axis=1, keepdims=True)) + m
        o_ref[...] = (z - lse)[:, :D_OUT]

    d0s, d1s = _deg_specs()
    return pl.pallas_call(
        body,
        grid=(NB,),
        in_specs=[
            pl.BlockSpec((1, B, 16), lambda r: (0, r, 0)),
            pl.BlockSpec((1, B, 16), lambda r: (1, r, 0)),
            pl.BlockSpec((B, 16), lambda r: (r, 0)),
            d0s, d1s,
            pl.BlockSpec((1, D_OUT), lambda r: (0, 0)),
        ],
        out_specs=pl.BlockSpec((B, D_OUT), lambda r: (r, 0)),
        out_shape=jax.ShapeDtypeStruct((N, D_OUT), jnp.float32),
    )(agg2, agg2, y2, deg, deg, b2p)


def kernel(features, edges, W1, b1, W2, b2):
    pad = E_PAD - E
    pad_idx = lax.iota(jnp.int32, pad)
    pad2 = jnp.stack([pad_idx % 4096, N + pad_idx % (NP - N)])
    edg4 = jnp.concatenate([edges, pad2], axis=1).reshape(2, NW, NBLK, K)

    z1 = jnp.zeros((RPT,), jnp.float32)
    z64 = jnp.zeros((RPT, D_HID), jnp.float32)
    z16 = jnp.zeros((RPT, 16), jnp.float32)
    b1r = b1.reshape(1, D_HID)
    b2r = b2.reshape(1, D_OUT)

    deg = _sc_degree(edg4, z1)[:, :, None]
    y1 = _tc1(features, W1, deg)
    agg1 = _sc_gather_scatter(y1, edg4, z64, D_HID)
    y2 = _tc2(agg1, y1, deg, b1r, W2)
    agg2 = _sc_gather_scatter(y2, edg4, z16, 16)
    return _tc3(agg2, y2, deg, b2r)
